# baseline (speedup 1.0000x reference)
import functools

import jax
import jax.numpy as jnp
from jax import lax
from jax.experimental import pallas as pl
from jax.experimental.pallas import tpu as pltpu
from jax.experimental.pallas import tpu_sc as plsc

_N = 10000
_E = 320000
_H = 64
_NC = 2
_NS = 16
_NW = _NC * _NS
_CHUNK = 128
_NCH = 79
_EPW = _NCH * _CHUNK
_EPAD = _NW * _EPW
_NACC = 10240
_RPW = _NACC // _NS
_DW = 16

_RB = 2000
_NBUF = 4


def _seg_body(y, ei, z64, out, src_v, dst_v, rows_v, acc_sh, sem_g, sem_s):
    c = lax.axis_index("c")
    s = lax.axis_index("s")
    wid = s * _NC + c
    row0 = s * _RPW

    pltpu.async_copy(z64.at[pl.ds(row0, _RPW)],
                     acc_sh.at[pl.ds(row0, _RPW)], sem_s.at[0])
    pltpu.async_copy(ei.at[0].at[wid], src_v, sem_g.at[0])
    pltpu.async_copy(ei.at[1].at[wid], dst_v, sem_g.at[1])
    pltpu.make_async_copy(ei.at[0].at[wid], src_v, sem_g.at[0]).wait()
    pltpu.make_async_copy(ei.at[1].at[wid], dst_v, sem_g.at[1]).wait()

    def gat(j, b):
        pltpu.async_copy(y.at[src_v.at[j]], rows_v.at[b], sem_g.at[b])

    def gat_wait(j, b):
        pltpu.make_async_copy(y.at[src_v.at[j]], rows_v.at[b],
                              sem_g.at[b]).wait()

    def sca(j, b):
        pltpu.async_copy(rows_v.at[b], acc_sh.at[dst_v.at[j]], sem_s.at[b],
                         add=True)

    def sca_wait(j, b):
        pltpu.make_async_copy(rows_v.at[b], acc_sh.at[dst_v.at[j]],
                              sem_s.at[b]).wait()

    gat(0, 0)
    gat(1, 1)
    pltpu.make_async_copy(z64.at[pl.ds(row0, _RPW)],
                          acc_sh.at[pl.ds(row0, _RPW)], sem_s.at[0]).wait()
    plsc.subcore_barrier()

    for j in (0, 1):
        gat_wait(j, j)
        sca(j, j)
        gat(j + 2, j + 2)
    for j in (2, 3):
        gat_wait(j, j)
        sca(j, j)
        sca_wait(j, j - 2)
        gat(j + 2, j - 2)

    _G = (_NCH - 7) // 4

    def group(g, carry):
        j0 = 4 + g * 4
        for b in range(4):
            j = j0 + b
            q = (b + 2) % 4
            gat_wait(j, b)
            sca(j, b)
            sca_wait(j, q)
            gat(j + 2, q)
        return carry

    lax.fori_loop(0, _G, group, 0)

    for j in range(4 + 4 * _G, _NCH):
        b = j % 4
        gat_wait(j, b)
        sca(j, b)
        if j + 2 < _NCH:
            sca_wait(j, (j + 2) % 4)
            gat(j + 2, (j + 2) % 4)

    for b in range(_NBUF):
        sca_wait(0, b)
    plsc.subcore_barrier()

    pltpu.sync_copy(acc_sh.at[pl.ds(row0, _RPW)],
                    out.at[c].at[pl.ds(row0, _RPW)])


def _deg_body(ei, z16, ones, deg_out, dst_v, ones_v, deg_sh, sem_g, sem_s):
    c = lax.axis_index("c")
    s = lax.axis_index("s")
    wid = s * _NC + c
    row0 = s * _RPW

    pltpu.async_copy(z16.at[pl.ds(row0, _RPW)],
                     deg_sh.at[pl.ds(row0, _RPW)], sem_s.at[0])
    pltpu.async_copy(ei.at[1].at[wid], dst_v, sem_g.at[0])
    pltpu.async_copy(ones, ones_v, sem_g.at[1])
    pltpu.make_async_copy(ei.at[1].at[wid], dst_v, sem_g.at[0]).wait()
    pltpu.make_async_copy(ones, ones_v, sem_g.at[1]).wait()
    pltpu.make_async_copy(z16.at[pl.ds(row0, _RPW)],
                          deg_sh.at[pl.ds(row0, _RPW)], sem_s.at[0]).wait()
    plsc.subcore_barrier()

    def chunk(j, carry):
        p = lax.rem(j, _NBUF)

        @pl.when(j >= _NBUF)
        def _():
            pltpu.make_async_copy(ones_v, deg_sh.at[dst_v.at[j]],
                                  sem_s.at[p]).wait()

        pltpu.async_copy(ones_v, deg_sh.at[dst_v.at[j]], sem_s.at[p],
                         add=True)
        return carry

    lax.fori_loop(0, _NCH, chunk, 0)
    for b in range(_NBUF):
        pltpu.make_async_copy(ones_v, deg_sh.at[dst_v.at[0]],
                              sem_s.at[b]).wait()
    plsc.subcore_barrier()
    pltpu.sync_copy(deg_sh.at[pl.ds(row0, _RPW)],
                    deg_out.at[c].at[pl.ds(row0, _RPW)])


def _sc_mesh():
    return plsc.VectorSubcoreMesh(core_axis_name="c", subcore_axis_name="s",
                                  num_cores=_NC, num_subcores=_NS)


def _make_seg():
    scratch = [
        pltpu.VMEM((_NCH, _CHUNK), jnp.int32),
        pltpu.VMEM((_NCH, _CHUNK), jnp.int32),
        pltpu.VMEM((_NBUF, _CHUNK, _H), jnp.float32),
        pltpu.VMEM_SHARED((_NACC, _H), jnp.float32),
        pltpu.SemaphoreType.DMA((_NBUF,)),
        pltpu.SemaphoreType.DMA((_NBUF,)),
    ]
    return pl.kernel(_seg_body,
                     out_type=jax.ShapeDtypeStruct((_NC, _NACC, _H),
                                                   jnp.float32),
                     mesh=_sc_mesh(), scratch_types=scratch,
                     compiler_params=pltpu.CompilerParams(
                         use_tc_tiling_on_sc=False))


def _make_deg():
    scratch = [
        pltpu.VMEM((_NCH, _CHUNK), jnp.int32),
        pltpu.VMEM((_CHUNK, _DW), jnp.float32),
        pltpu.VMEM_SHARED((_NACC, _DW), jnp.float32),
        pltpu.SemaphoreType.DMA((_NBUF,)),
        pltpu.SemaphoreType.DMA((_NBUF,)),
    ]
    return pl.kernel(_deg_body,
                     out_type=jax.ShapeDtypeStruct((_NC, _NACC, _DW),
                                                   jnp.float32),
                     mesh=_sc_mesh(), scratch_types=scratch,
                     compiler_params=pltpu.CompilerParams(
                         use_tc_tiling_on_sc=False))


_seg_fns = {}


def _deg(*args):
    if 'deg' not in _seg_fns:
        _seg_fns['deg'] = _make_deg()
    return _seg_fns['deg'](*args)


def _seg(*args):
    if 'seg' not in _seg_fns:
        _seg_fns['seg'] = _make_seg()
    return _seg_fns['seg'](*args)


def _mmp(a, w):
    return lax.dot_general(a, w, (((1,), (0,)), ((), ())),
                           preferred_element_type=jnp.float32,
                           precision=lax.Precision.HIGHEST)


def _tc0_body(x_r, wl_r, wr_r, y_r, r_r):
    x = x_r[...]
    y_r[...] = _mmp(x, wl_r[...])
    r_r[...] = _mmp(x, wr_r[...])


def _mid_body(acc_r, rec_r, r_r, bl_r, wln_r, wrn_r, y_r, rn_r):
    a = acc_r[...]
    h = jnp.maximum((a[0] + a[1]) * rec_r[...] + bl_r[...] + r_r[...], 0.0)
    y_r[...] = _mmp(h, wln_r[...])
    rn_r[...] = _mmp(h, wrn_r[...])


def _fin_body(acc_r, rec_r, r_r, bl_r, wc1_r, bc1_r, wc2_r, bc2_r, o_r):
    a = acc_r[...]
    h = jnp.maximum((a[0] + a[1]) * rec_r[...] + bl_r[...] + r_r[...], 0.0)
    z = jnp.maximum(_mmp(h, wc1_r[...]) + bc1_r[...], 0.0)
    o_r[...] = _mmp(z, wc2_r[...]) + bc2_r[...]


_RBP = _RB // 2
_NP = _N // 2


def _pk_spec(width):
    return pl.BlockSpec((_RBP, width), lambda i: (i, 0))


def _accp_spec():
    return pl.BlockSpec((_NC, _RBP, 2 * _H), lambda i: (0, i, 0))


def _full_spec(shape):
    return pl.BlockSpec(shape, lambda i: tuple(0 for _ in shape))


_GRID = _N // _RB

_tc0 = pl.pallas_call(
    _tc0_body,
    grid=(_GRID,),
    in_specs=[pl.BlockSpec((_RB, 128), lambda i: (i, 0)),
              _full_spec((128, _H)), _full_spec((128, _H))],
    out_specs=[pl.BlockSpec((_RB, _H), lambda i: (i, 0))] * 2,
    out_shape=[jax.ShapeDtypeStruct((_N, _H), jnp.float32)] * 2,
)

_mid = pl.pallas_call(
    _mid_body,
    grid=(_GRID,),
    in_specs=[_accp_spec(), _pk_spec(2 * _H), _pk_spec(2 * _H),
              _full_spec((1, 2 * _H)), _full_spec((2 * _H, 2 * _H)),
              _full_spec((2 * _H, 2 * _H))],
    out_specs=[_pk_spec(2 * _H), _pk_spec(2 * _H)],
    out_shape=[jax.ShapeDtypeStruct((_NP, 2 * _H), jnp.float32)] * 2,
)

_fin = pl.pallas_call(
    _fin_body,
    grid=(_GRID,),
    in_specs=[_accp_spec(), _pk_spec(2 * _H), _pk_spec(2 * _H),
              _full_spec((1, 2 * _H)), _full_spec((2 * _H, _H)),
              _full_spec((1, _H)), _full_spec((_H, 94)),
              _full_spec((1, 94))],
    out_specs=_pk_spec(94),
    out_shape=jax.ShapeDtypeStruct((_NP, 94), jnp.float32),
)


def _bd(w):
    wt = w.T
    z = jnp.zeros_like(wt)
    return jnp.concatenate(
        [jnp.concatenate([wt, z], axis=1), jnp.concatenate([z, wt], axis=1)],
        axis=0)


def _bpk(b):
    return jnp.concatenate([b, b]).reshape(1, -1)


def kernel(x, edge_index, Wl11, bl11, Wr11, Wl12, bl12, Wr12,
           Wl21, bl21, Wr21, Wl22, bl22, Wr22, Wc1, bc1, Wc2, bc2):
    src = edge_index[0]
    dst = edge_index[1]
    npad = _EPAD - _E
    it = jnp.arange(npad, dtype=jnp.int32)
    ei_p = jnp.concatenate(
        [edge_index,
         jnp.stack([it % _N, _N + it % (_NACC - _N)])], axis=1).reshape(
             2, _NW, _NCH, _CHUNK)
    z64 = jnp.zeros((_NACC, _H), jnp.float32)
    z16 = jnp.zeros((_NACC, _DW), jnp.float32)
    ones = jnp.zeros((_CHUNK, _DW), jnp.float32).at[:, 0].set(1.0)

    def pk(a):
        return a.reshape(_NC, _NACC // 2, 2 * _H)

    def unpk(ypk):
        return ypk.reshape(_N, _H)

    dega = _deg(ei_p, z16, ones)
    y1, r1 = _tc0(x, Wl11.T, Wr11.T)
    r1 = r1.reshape(_NP, 2 * _H)
    acc1 = _seg(y1, ei_p, z64)
    ds = dega[0, :, 0] + dega[1, :, 0]
    rec = (1.0 / jnp.maximum(ds, 1.0)).reshape(_NACC // 2, 2, 1)
    rec = jnp.broadcast_to(rec, (_NACC // 2, 2, _H)).reshape(
        _NACC // 2, 2 * _H)

    y2, r2 = _mid(pk(acc1), rec, r1, _bpk(bl11), _bd(Wl12), _bd(Wr12))
    acc2 = _seg(unpk(y2), ei_p, z64)
    y3, r3 = _mid(pk(acc2), rec, r2, _bpk(bl12), _bd(Wl21), _bd(Wr21))
    acc3 = _seg(unpk(y3), ei_p, z64)
    y4, r4 = _mid(pk(acc3), rec, r3, _bpk(bl21), _bd(Wl22), _bd(Wr22))
    acc4 = _seg(unpk(y4), ei_p, z64)
    out = _fin(pk(acc4), rec, r4, _bpk(bl22),
               _bd(Wc1), _bpk(bc1), _bd(Wc2), _bpk(bc2))
    return out.reshape(_N, 47)


# speedup vs baseline: 1.1348x; 1.1348x over previous
import functools

import jax
import jax.numpy as jnp
from jax import lax
from jax.experimental import pallas as pl
from jax.experimental.pallas import tpu as pltpu
from jax.experimental.pallas import tpu_sc as plsc

_N = 10000
_E = 320000
_H = 64
_NC = 2
_NS = 16
_NW = _NC * _NS
_CHUNK = 256
_NCH = 40
_EPW = _NCH * _CHUNK
_EPAD = _NW * _EPW
_NACC = 10240
_RPW = _NACC // _NS
_DW = 16

_RB = 2000
_NBUF = 4


def _seg_body(y, ei, z64, out, src_v, dst_v, rows_v, acc_sh, sem_g, sem_s):
    c = lax.axis_index("c")
    s = lax.axis_index("s")
    wid = s * _NC + c
    row0 = s * _RPW

    pltpu.async_copy(z64.at[pl.ds(row0, _RPW)],
                     acc_sh.at[pl.ds(row0, _RPW)], sem_s.at[0])
    pltpu.async_copy(ei.at[0].at[wid], src_v, sem_g.at[0])
    pltpu.async_copy(ei.at[1].at[wid], dst_v, sem_g.at[1])
    pltpu.make_async_copy(ei.at[0].at[wid], src_v, sem_g.at[0]).wait()
    pltpu.make_async_copy(ei.at[1].at[wid], dst_v, sem_g.at[1]).wait()

    def gat(j, b):
        pltpu.async_copy(y.at[src_v.at[j]], rows_v.at[b], sem_g.at[b])

    def gat_wait(j, b):
        pltpu.make_async_copy(y.at[src_v.at[j]], rows_v.at[b],
                              sem_g.at[b]).wait()

    def sca(j, b):
        pltpu.async_copy(rows_v.at[b], acc_sh.at[dst_v.at[j]], sem_s.at[b],
                         add=True)

    def sca_wait(j, b):
        pltpu.make_async_copy(rows_v.at[b], acc_sh.at[dst_v.at[j]],
                              sem_s.at[b]).wait()

    gat(0, 0)
    gat(1, 1)
    pltpu.make_async_copy(z64.at[pl.ds(row0, _RPW)],
                          acc_sh.at[pl.ds(row0, _RPW)], sem_s.at[0]).wait()
    plsc.subcore_barrier()

    for j in (0, 1):
        gat_wait(j, j)
        sca(j, j)
        gat(j + 2, j + 2)
    for j in (2, 3):
        gat_wait(j, j)
        sca(j, j)
        sca_wait(j, j - 2)
        gat(j + 2, j - 2)

    _G = (_NCH - 7) // 4

    def group(g, carry):
        j0 = 4 + g * 4
        for b in range(4):
            j = j0 + b
            q = (b + 2) % 4
            gat_wait(j, b)
            sca(j, b)
            sca_wait(j, q)
            gat(j + 2, q)
        return carry

    lax.fori_loop(0, _G, group, 0)

    for j in range(4 + 4 * _G, _NCH):
        b = j % 4
        gat_wait(j, b)
        sca(j, b)
        if j + 2 < _NCH:
            sca_wait(j, (j + 2) % 4)
            gat(j + 2, (j + 2) % 4)

    for b in range(_NBUF):
        sca_wait(0, b)
    plsc.subcore_barrier()

    pltpu.sync_copy(acc_sh.at[pl.ds(row0, _RPW)],
                    out.at[c].at[pl.ds(row0, _RPW)])


def _deg_body(ei, z16, ones, deg_out, dst_v, ones_v, deg_sh, sem_g, sem_s):
    c = lax.axis_index("c")
    s = lax.axis_index("s")
    wid = s * _NC + c
    row0 = s * _RPW

    pltpu.async_copy(z16.at[pl.ds(row0, _RPW)],
                     deg_sh.at[pl.ds(row0, _RPW)], sem_s.at[0])
    pltpu.async_copy(ei.at[1].at[wid], dst_v, sem_g.at[0])
    pltpu.async_copy(ones, ones_v, sem_g.at[1])
    pltpu.make_async_copy(ei.at[1].at[wid], dst_v, sem_g.at[0]).wait()
    pltpu.make_async_copy(ones, ones_v, sem_g.at[1]).wait()
    pltpu.make_async_copy(z16.at[pl.ds(row0, _RPW)],
                          deg_sh.at[pl.ds(row0, _RPW)], sem_s.at[0]).wait()
    plsc.subcore_barrier()

    def chunk(j, carry):
        p = lax.rem(j, _NBUF)

        @pl.when(j >= _NBUF)
        def _():
            pltpu.make_async_copy(ones_v, deg_sh.at[dst_v.at[j]],
                                  sem_s.at[p]).wait()

        pltpu.async_copy(ones_v, deg_sh.at[dst_v.at[j]], sem_s.at[p],
                         add=True)
        return carry

    lax.fori_loop(0, _NCH, chunk, 0)
    for b in range(_NBUF):
        pltpu.make_async_copy(ones_v, deg_sh.at[dst_v.at[0]],
                              sem_s.at[b]).wait()
    plsc.subcore_barrier()
    pltpu.sync_copy(deg_sh.at[pl.ds(row0, _RPW)],
                    deg_out.at[c].at[pl.ds(row0, _RPW)])


def _sc_mesh():
    return plsc.VectorSubcoreMesh(core_axis_name="c", subcore_axis_name="s",
                                  num_cores=_NC, num_subcores=_NS)


def _make_seg():
    scratch = [
        pltpu.VMEM((_NCH, _CHUNK), jnp.int32),
        pltpu.VMEM((_NCH, _CHUNK), jnp.int32),
        pltpu.VMEM((_NBUF, _CHUNK, _H), jnp.float32),
        pltpu.VMEM_SHARED((_NACC, _H), jnp.float32),
        pltpu.SemaphoreType.DMA((_NBUF,)),
        pltpu.SemaphoreType.DMA((_NBUF,)),
    ]
    return pl.kernel(_seg_body,
                     out_type=jax.ShapeDtypeStruct((_NC, _NACC, _H),
                                                   jnp.float32),
                     mesh=_sc_mesh(), scratch_types=scratch,
                     compiler_params=pltpu.CompilerParams(
                         use_tc_tiling_on_sc=False))


def _make_deg():
    scratch = [
        pltpu.VMEM((_NCH, _CHUNK), jnp.int32),
        pltpu.VMEM((_CHUNK, _DW), jnp.float32),
        pltpu.VMEM_SHARED((_NACC, _DW), jnp.float32),
        pltpu.SemaphoreType.DMA((_NBUF,)),
        pltpu.SemaphoreType.DMA((_NBUF,)),
    ]
    return pl.kernel(_deg_body,
                     out_type=jax.ShapeDtypeStruct((_NC, _NACC, _DW),
                                                   jnp.float32),
                     mesh=_sc_mesh(), scratch_types=scratch,
                     compiler_params=pltpu.CompilerParams(
                         use_tc_tiling_on_sc=False))


_seg_fns = {}


def _deg(*args):
    if 'deg' not in _seg_fns:
        _seg_fns['deg'] = _make_deg()
    return _seg_fns['deg'](*args)


def _seg(*args):
    if 'seg' not in _seg_fns:
        _seg_fns['seg'] = _make_seg()
    return _seg_fns['seg'](*args)


def _mmp(a, w):
    return lax.dot_general(a, w, (((1,), (0,)), ((), ())),
                           preferred_element_type=jnp.float32)


def _tc0_body(x_r, wl_r, wr_r, y_r, r_r):
    x = x_r[...]
    y_r[...] = _mmp(x, wl_r[...])
    r_r[...] = _mmp(x, wr_r[...])


def _mid_body(acc_r, rec_r, r_r, bl_r, wln_r, wrn_r, y_r, rn_r):
    a = acc_r[...]
    h = jnp.maximum((a[0] + a[1]) * rec_r[...] + bl_r[...] + r_r[...], 0.0)
    y_r[...] = _mmp(h, wln_r[...])
    rn_r[...] = _mmp(h, wrn_r[...])


def _fin_body(acc_r, rec_r, r_r, bl_r, wc1_r, bc1_r, wc2_r, bc2_r, o_r):
    a = acc_r[...]
    h = jnp.maximum((a[0] + a[1]) * rec_r[...] + bl_r[...] + r_r[...], 0.0)
    z = jnp.maximum(_mmp(h, wc1_r[...]) + bc1_r[...], 0.0)
    o_r[...] = _mmp(z, wc2_r[...]) + bc2_r[...]


_RBP = _RB // 2
_NP = _N // 2


def _pk_spec(width):
    return pl.BlockSpec((_RBP, width), lambda i: (i, 0))


def _accp_spec():
    return pl.BlockSpec((_NC, _RBP, 2 * _H), lambda i: (0, i, 0))


def _full_spec(shape):
    return pl.BlockSpec(shape, lambda i: tuple(0 for _ in shape))


_GRID = _N // _RB

_tc0 = pl.pallas_call(
    _tc0_body,
    grid=(_GRID,),
    in_specs=[pl.BlockSpec((_RB, 128), lambda i: (i, 0)),
              _full_spec((128, _H)), _full_spec((128, _H))],
    out_specs=[pl.BlockSpec((_RB, _H), lambda i: (i, 0))] * 2,
    out_shape=[jax.ShapeDtypeStruct((_N, _H), jnp.float32)] * 2,
)

_mid = pl.pallas_call(
    _mid_body,
    grid=(_GRID,),
    in_specs=[_accp_spec(), _pk_spec(2 * _H), _pk_spec(2 * _H),
              _full_spec((1, 2 * _H)), _full_spec((2 * _H, 2 * _H)),
              _full_spec((2 * _H, 2 * _H))],
    out_specs=[_pk_spec(2 * _H), _pk_spec(2 * _H)],
    out_shape=[jax.ShapeDtypeStruct((_NP, 2 * _H), jnp.float32)] * 2,
)

_fin = pl.pallas_call(
    _fin_body,
    grid=(_GRID,),
    in_specs=[_accp_spec(), _pk_spec(2 * _H), _pk_spec(2 * _H),
              _full_spec((1, 2 * _H)), _full_spec((2 * _H, _H)),
              _full_spec((1, _H)), _full_spec((_H, 94)),
              _full_spec((1, 94))],
    out_specs=_pk_spec(94),
    out_shape=jax.ShapeDtypeStruct((_NP, 94), jnp.float32),
)


def _bd(w):
    wt = w.T
    z = jnp.zeros_like(wt)
    return jnp.concatenate(
        [jnp.concatenate([wt, z], axis=1), jnp.concatenate([z, wt], axis=1)],
        axis=0)


def _bpk(b):
    return jnp.concatenate([b, b]).reshape(1, -1)


def kernel(x, edge_index, Wl11, bl11, Wr11, Wl12, bl12, Wr12,
           Wl21, bl21, Wr21, Wl22, bl22, Wr22, Wc1, bc1, Wc2, bc2):
    src = edge_index[0]
    dst = edge_index[1]
    npad = _EPAD - _E
    it = jnp.arange(npad, dtype=jnp.int32)
    ei_p = jnp.concatenate(
        [edge_index,
         jnp.stack([it % _N, _N + it % (_NACC - _N)])], axis=1).reshape(
             2, _NW, _NCH, _CHUNK)
    z64 = jnp.zeros((_NACC, _H), jnp.float32)
    z16 = jnp.zeros((_NACC, _DW), jnp.float32)
    ones = jnp.zeros((_CHUNK, _DW), jnp.float32).at[:, 0].set(1.0)

    def pk(a):
        return a.reshape(_NC, _NACC // 2, 2 * _H)

    def unpk(ypk):
        return ypk.reshape(_N, _H)

    dega = _deg(ei_p, z16, ones)
    y1, r1 = _tc0(x, Wl11.T, Wr11.T)
    r1 = r1.reshape(_NP, 2 * _H)
    acc1 = _seg(y1, ei_p, z64)
    ds = dega[0, :, 0] + dega[1, :, 0]
    rec = (1.0 / jnp.maximum(ds, 1.0)).reshape(_NACC // 2, 2, 1)
    rec = jnp.broadcast_to(rec, (_NACC // 2, 2, _H)).reshape(
        _NACC // 2, 2 * _H)

    y2, r2 = _mid(pk(acc1), rec, r1, _bpk(bl11), _bd(Wl12), _bd(Wr12))
    acc2 = _seg(unpk(y2), ei_p, z64)
    y3, r3 = _mid(pk(acc2), rec, r2, _bpk(bl12), _bd(Wl21), _bd(Wr21))
    acc3 = _seg(unpk(y3), ei_p, z64)
    y4, r4 = _mid(pk(acc3), rec, r3, _bpk(bl21), _bd(Wl22), _bd(Wr22))
    acc4 = _seg(unpk(y4), ei_p, z64)
    out = _fin(pk(acc4), rec, r4, _bpk(bl22),
               _bd(Wc1), _bpk(bc1), _bd(Wc2), _bpk(bc2))
    return out.reshape(_N, 47)


# speedup vs baseline: 1.1361x; 1.0012x over previous
import functools

import jax
import jax.numpy as jnp
from jax import lax
from jax.experimental import pallas as pl
from jax.experimental.pallas import tpu as pltpu
from jax.experimental.pallas import tpu_sc as plsc

_N = 10000
_E = 320000
_H = 64
_NC = 2
_NS = 16
_NW = _NC * _NS
_CHUNK = 256
_NCH = 40
_EPW = _NCH * _CHUNK
_EPAD = _NW * _EPW
_NACC = 10240
_RPW = _NACC // _NS
_DW = 16

_RB = 2000
_NBUF = 4


def _seg_body(y, ei, z64, out, src_v, dst_v, rows_v, acc_sh, sem_g, sem_s):
    c = lax.axis_index("c")
    s = lax.axis_index("s")
    wid = s * _NC + c
    row0 = s * _RPW

    pltpu.async_copy(z64.at[pl.ds(row0, _RPW)],
                     acc_sh.at[pl.ds(row0, _RPW)], sem_s.at[0])
    pltpu.async_copy(ei.at[0].at[wid], src_v, sem_g.at[0])
    pltpu.async_copy(ei.at[1].at[wid], dst_v, sem_g.at[1])
    pltpu.make_async_copy(ei.at[0].at[wid], src_v, sem_g.at[0]).wait()
    pltpu.make_async_copy(ei.at[1].at[wid], dst_v, sem_g.at[1]).wait()

    def gat(j, b):
        pltpu.async_copy(y.at[src_v.at[j]], rows_v.at[b], sem_g.at[b])

    def gat_wait(j, b):
        pltpu.make_async_copy(y.at[src_v.at[j]], rows_v.at[b],
                              sem_g.at[b]).wait()

    def sca(j, b):
        pltpu.async_copy(rows_v.at[b], acc_sh.at[dst_v.at[j]], sem_s.at[b],
                         add=True)

    def sca_wait(j, b):
        pltpu.make_async_copy(rows_v.at[b], acc_sh.at[dst_v.at[j]],
                              sem_s.at[b]).wait()

    gat(0, 0)
    gat(1, 1)
    pltpu.make_async_copy(z64.at[pl.ds(row0, _RPW)],
                          acc_sh.at[pl.ds(row0, _RPW)], sem_s.at[0]).wait()
    plsc.subcore_barrier()

    for j in (0, 1):
        gat_wait(j, j)
        sca(j, j)
        gat(j + 2, j + 2)
    for j in (2, 3):
        gat_wait(j, j)
        sca(j, j)
        sca_wait(j, j - 2)
        gat(j + 2, j - 2)

    _G = (_NCH - 7) // 4

    def group(g, carry):
        j0 = 4 + g * 4
        for b in range(4):
            j = j0 + b
            q = (b + 2) % 4
            gat_wait(j, b)
            sca_wait(j, q)
            gat(j + 2, q)
            sca(j, b)
        return carry

    lax.fori_loop(0, _G, group, 0)

    for j in range(4 + 4 * _G, _NCH):
        b = j % 4
        gat_wait(j, b)
        sca(j, b)
        if j + 2 < _NCH:
            sca_wait(j, (j + 2) % 4)
            gat(j + 2, (j + 2) % 4)

    for b in range(_NBUF):
        sca_wait(0, b)
    plsc.subcore_barrier()

    pltpu.sync_copy(acc_sh.at[pl.ds(row0, _RPW)],
                    out.at[c].at[pl.ds(row0, _RPW)])


def _deg_body(ei, z16, ones, deg_out, dst_v, ones_v, deg_sh, sem_g, sem_s):
    c = lax.axis_index("c")
    s = lax.axis_index("s")
    wid = s * _NC + c
    row0 = s * _RPW

    pltpu.async_copy(z16.at[pl.ds(row0, _RPW)],
                     deg_sh.at[pl.ds(row0, _RPW)], sem_s.at[0])
    pltpu.async_copy(ei.at[1].at[wid], dst_v, sem_g.at[0])
    pltpu.async_copy(ones, ones_v, sem_g.at[1])
    pltpu.make_async_copy(ei.at[1].at[wid], dst_v, sem_g.at[0]).wait()
    pltpu.make_async_copy(ones, ones_v, sem_g.at[1]).wait()
    pltpu.make_async_copy(z16.at[pl.ds(row0, _RPW)],
                          deg_sh.at[pl.ds(row0, _RPW)], sem_s.at[0]).wait()
    plsc.subcore_barrier()

    def chunk(j, carry):
        p = lax.rem(j, _NBUF)

        @pl.when(j >= _NBUF)
        def _():
            pltpu.make_async_copy(ones_v, deg_sh.at[dst_v.at[j]],
                                  sem_s.at[p]).wait()

        pltpu.async_copy(ones_v, deg_sh.at[dst_v.at[j]], sem_s.at[p],
                         add=True)
        return carry

    lax.fori_loop(0, _NCH, chunk, 0)
    for b in range(_NBUF):
        pltpu.make_async_copy(ones_v, deg_sh.at[dst_v.at[0]],
                              sem_s.at[b]).wait()
    plsc.subcore_barrier()
    pltpu.sync_copy(deg_sh.at[pl.ds(row0, _RPW)],
                    deg_out.at[c].at[pl.ds(row0, _RPW)])


def _sc_mesh():
    return plsc.VectorSubcoreMesh(core_axis_name="c", subcore_axis_name="s",
                                  num_cores=_NC, num_subcores=_NS)


def _make_seg():
    scratch = [
        pltpu.VMEM((_NCH, _CHUNK), jnp.int32),
        pltpu.VMEM((_NCH, _CHUNK), jnp.int32),
        pltpu.VMEM((_NBUF, _CHUNK, _H), jnp.float32),
        pltpu.VMEM_SHARED((_NACC, _H), jnp.float32),
        pltpu.SemaphoreType.DMA((_NBUF,)),
        pltpu.SemaphoreType.DMA((_NBUF,)),
    ]
    return pl.kernel(_seg_body,
                     out_type=jax.ShapeDtypeStruct((_NC, _NACC, _H),
                                                   jnp.float32),
                     mesh=_sc_mesh(), scratch_types=scratch,
                     compiler_params=pltpu.CompilerParams(
                         use_tc_tiling_on_sc=False))


def _make_deg():
    scratch = [
        pltpu.VMEM((_NCH, _CHUNK), jnp.int32),
        pltpu.VMEM((_CHUNK, _DW), jnp.float32),
        pltpu.VMEM_SHARED((_NACC, _DW), jnp.float32),
        pltpu.SemaphoreType.DMA((_NBUF,)),
        pltpu.SemaphoreType.DMA((_NBUF,)),
    ]
    return pl.kernel(_deg_body,
                     out_type=jax.ShapeDtypeStruct((_NC, _NACC, _DW),
                                                   jnp.float32),
                     mesh=_sc_mesh(), scratch_types=scratch,
                     compiler_params=pltpu.CompilerParams(
                         use_tc_tiling_on_sc=False))


_seg_fns = {}


def _deg(*args):
    if 'deg' not in _seg_fns:
        _seg_fns['deg'] = _make_deg()
    return _seg_fns['deg'](*args)


def _seg(*args):
    if 'seg' not in _seg_fns:
        _seg_fns['seg'] = _make_seg()
    return _seg_fns['seg'](*args)


def _mmp(a, w):
    return lax.dot_general(a, w, (((1,), (0,)), ((), ())),
                           preferred_element_type=jnp.float32)


def _tc0_body(x_r, wl_r, wr_r, y_r, r_r):
    x = x_r[...]
    y_r[...] = _mmp(x, wl_r[...])
    r_r[...] = _mmp(x, wr_r[...])


def _mid_body(acc_r, rec_r, r_r, bl_r, wln_r, wrn_r, y_r, rn_r):
    a = acc_r[...]
    h = jnp.maximum((a[0] + a[1]) * rec_r[...] + bl_r[...] + r_r[...], 0.0)
    y_r[...] = _mmp(h, wln_r[...])
    rn_r[...] = _mmp(h, wrn_r[...])


def _fin_body(acc_r, rec_r, r_r, bl_r, wc1_r, bc1_r, wc2_r, bc2_r, o_r):
    a = acc_r[...]
    h = jnp.maximum((a[0] + a[1]) * rec_r[...] + bl_r[...] + r_r[...], 0.0)
    z = jnp.maximum(_mmp(h, wc1_r[...]) + bc1_r[...], 0.0)
    o_r[...] = _mmp(z, wc2_r[...]) + bc2_r[...]


_RBP = _RB // 2
_NP = _N // 2


def _pk_spec(width):
    return pl.BlockSpec((_RBP, width), lambda i: (i, 0))


def _accp_spec():
    return pl.BlockSpec((_NC, _RBP, 2 * _H), lambda i: (0, i, 0))


def _full_spec(shape):
    return pl.BlockSpec(shape, lambda i: tuple(0 for _ in shape))


_GRID = _N // _RB

_tc0 = pl.pallas_call(
    _tc0_body,
    grid=(_GRID,),
    in_specs=[pl.BlockSpec((_RB, 128), lambda i: (i, 0)),
              _full_spec((128, _H)), _full_spec((128, _H))],
    out_specs=[pl.BlockSpec((_RB, _H), lambda i: (i, 0))] * 2,
    out_shape=[jax.ShapeDtypeStruct((_N, _H), jnp.float32)] * 2,
)

_mid = pl.pallas_call(
    _mid_body,
    grid=(_GRID,),
    in_specs=[_accp_spec(), _pk_spec(2 * _H), _pk_spec(2 * _H),
              _full_spec((1, 2 * _H)), _full_spec((2 * _H, 2 * _H)),
              _full_spec((2 * _H, 2 * _H))],
    out_specs=[_pk_spec(2 * _H), _pk_spec(2 * _H)],
    out_shape=[jax.ShapeDtypeStruct((_NP, 2 * _H), jnp.float32)] * 2,
)

_fin = pl.pallas_call(
    _fin_body,
    grid=(_GRID,),
    in_specs=[_accp_spec(), _pk_spec(2 * _H), _pk_spec(2 * _H),
              _full_spec((1, 2 * _H)), _full_spec((2 * _H, _H)),
              _full_spec((1, _H)), _full_spec((_H, 94)),
              _full_spec((1, 94))],
    out_specs=_pk_spec(94),
    out_shape=jax.ShapeDtypeStruct((_NP, 94), jnp.float32),
)


def _bd(w):
    wt = w.T
    z = jnp.zeros_like(wt)
    return jnp.concatenate(
        [jnp.concatenate([wt, z], axis=1), jnp.concatenate([z, wt], axis=1)],
        axis=0)


def _bpk(b):
    return jnp.concatenate([b, b]).reshape(1, -1)


def kernel(x, edge_index, Wl11, bl11, Wr11, Wl12, bl12, Wr12,
           Wl21, bl21, Wr21, Wl22, bl22, Wr22, Wc1, bc1, Wc2, bc2):
    src = edge_index[0]
    dst = edge_index[1]
    npad = _EPAD - _E
    it = jnp.arange(npad, dtype=jnp.int32)
    ei_p = jnp.concatenate(
        [edge_index,
         jnp.stack([it % _N, _N + it % (_NACC - _N)])], axis=1).reshape(
             2, _NW, _NCH, _CHUNK)
    z64 = jnp.zeros((_NACC, _H), jnp.float32)
    z16 = jnp.zeros((_NACC, _DW), jnp.float32)
    ones = jnp.zeros((_CHUNK, _DW), jnp.float32).at[:, 0].set(1.0)

    def pk(a):
        return a.reshape(_NC, _NACC // 2, 2 * _H)

    def unpk(ypk):
        return ypk.reshape(_N, _H)

    dega = _deg(ei_p, z16, ones)
    y1, r1 = _tc0(x, Wl11.T, Wr11.T)
    r1 = r1.reshape(_NP, 2 * _H)
    acc1 = _seg(y1, ei_p, z64)
    ds = dega[0, :, 0] + dega[1, :, 0]
    rec = (1.0 / jnp.maximum(ds, 1.0)).reshape(_NACC // 2, 2, 1)
    rec = jnp.broadcast_to(rec, (_NACC // 2, 2, _H)).reshape(
        _NACC // 2, 2 * _H)

    y2, r2 = _mid(pk(acc1), rec, r1, _bpk(bl11), _bd(Wl12), _bd(Wr12))
    acc2 = _seg(unpk(y2), ei_p, z64)
    y3, r3 = _mid(pk(acc2), rec, r2, _bpk(bl12), _bd(Wl21), _bd(Wr21))
    acc3 = _seg(unpk(y3), ei_p, z64)
    y4, r4 = _mid(pk(acc3), rec, r3, _bpk(bl21), _bd(Wl22), _bd(Wr22))
    acc4 = _seg(unpk(y4), ei_p, z64)
    out = _fin(pk(acc4), rec, r4, _bpk(bl22),
               _bd(Wc1), _bpk(bc1), _bd(Wc2), _bpk(bc2))
    return out.reshape(_N, 47)
